# baseline (speedup 1.0000x reference)
import functools

import jax
import jax.numpy as jnp
from jax import lax
from jax.experimental import pallas as pl
from jax.experimental.pallas import tpu as pltpu
from jax.experimental.pallas import tpu_sc as plsc

_D = 64
_L = 200
_B = 4096
_NW = 32
_BPW = _B // _NW
_TD = _D // 8


def _transpose_add(l, rows_b, tile_b, pos_v):
    ivec = lax.iota(jnp.int32, 16)

    def td_body(td, carry):
        bls = [ivec + (16 * g) for g in range(8)]
        lvec = jnp.full((16,), l, jnp.int32)
        for dl in range(8):
            d = td * 8 + dl
            col = jnp.full((16,), d, jnp.int32)
            pvec = plsc.load_gather(pos_v, [lvec, col])
            for g in range(8):
                vals = plsc.load_gather(rows_b, [bls[g], col])
                tile_b[td, dl, pl.ds(g * 16, 16)] = vals + pvec
        return carry

    lax.fori_loop(0, _TD, td_body, 0)


def _sc_body(xt_hbm, tok_hbm, pos_hbm, out_hbm,
             pos_v, idx0, idx1, rows0, rows1, tile0, tile1,
             isem0, isem1, gsem0, gsem1, wsem0, wsem1):
    wid = lax.axis_index("s") * 2 + lax.axis_index("c")
    b0 = wid * _BPW

    bufs = ((idx0, rows0, tile0, isem0, gsem0, wsem0),
            (idx1, rows1, tile1, isem1, gsem1, wsem1))

    pltpu.sync_copy(pos_hbm, pos_v)

    def idx_start(l, idx_b, isem_b):
        pltpu.async_copy(xt_hbm.at[l, pl.ds(b0, _BPW)], idx_b, isem_b)

    def idx_wait(l, idx_b, isem_b):
        pltpu.make_async_copy(
            xt_hbm.at[l, pl.ds(b0, _BPW)], idx_b, isem_b).wait()

    def gather_start(idx_b, rows_b, gsem_b):
        pltpu.async_copy(tok_hbm.at[idx_b], rows_b, gsem_b)

    def gather_wait(idx_b, rows_b, gsem_b):
        pltpu.make_async_copy(tok_hbm.at[idx_b], rows_b, gsem_b).wait()

    def write_start(l, tile_b, wsem_b):
        pltpu.async_copy(tile_b, out_hbm.at[l, :, wid], wsem_b)

    def write_wait(l, tile_b, wsem_b):
        pltpu.make_async_copy(tile_b, out_hbm.at[l, :, wid], wsem_b).wait()

    idx_start(0, idx0, isem0)
    idx_wait(0, idx0, isem0)
    gather_start(idx0, rows0, gsem0)
    idx_start(1, idx1, isem1)

    def outer(i, carry):
        for b in range(2):
            l = i * 2 + b
            idx_b, rows_b, tile_b, isem_b, gsem_b, wsem_b = bufs[b]
            idx_n, rows_n, tile_n, isem_n, gsem_n, wsem_n = bufs[1 - b]

            @pl.when(l + 1 < _L)
            def _start_next_gather():
                idx_wait(l + 1, idx_n, isem_n)
                gather_start(idx_n, rows_n, gsem_n)

            gather_wait(idx_b, rows_b, gsem_b)

            @pl.when(l + 2 < _L)
            def _start_next_idx():
                idx_start(l + 2, idx_b, isem_b)

            @pl.when(l >= 2)
            def _drain_write():
                write_wait(l - 2, tile_b, wsem_b)

            _transpose_add(l, rows_b, tile_b, pos_v)
            write_start(l, tile_b, wsem_b)
        return carry

    lax.fori_loop(0, _L // 2, outer, 0)

    write_wait(_L - 2, tile0, wsem0)
    write_wait(_L - 1, tile1, wsem1)


@jax.jit
def _sc_embed(x, token_table, pos_table):
    mesh = plsc.VectorSubcoreMesh(core_axis_name="c", subcore_axis_name="s")
    out5d = pl.kernel(
        _sc_body,
        mesh=mesh,
        out_type=jax.ShapeDtypeStruct((_L, _TD, _NW, 8, _BPW), jnp.float32),
        compiler_params=pltpu.CompilerParams(
            use_tc_tiling_on_sc=False, needs_layout_passes=False),
        scratch_types=[
            pltpu.VMEM((_L, _D), jnp.float32),
            pltpu.VMEM((_BPW,), jnp.int32),
            pltpu.VMEM((_BPW,), jnp.int32),
            pltpu.VMEM((_BPW, _D), jnp.float32),
            pltpu.VMEM((_BPW, _D), jnp.float32),
            pltpu.VMEM((_TD, 8, _BPW), jnp.float32),
            pltpu.VMEM((_TD, 8, _BPW), jnp.float32),
            pltpu.SemaphoreType.DMA,
            pltpu.SemaphoreType.DMA,
            pltpu.SemaphoreType.DMA,
            pltpu.SemaphoreType.DMA,
            pltpu.SemaphoreType.DMA,
            pltpu.SemaphoreType.DMA,
        ],
    )(x.T, token_table, pos_table)
    return out5d.transpose(2, 4, 0, 1, 3).reshape(_B, _L, _D)


def kernel(x, token_table, pos_table):
    return _sc_embed(x, token_table, pos_table)


# speedup vs baseline: 2.0461x; 2.0461x over previous
import functools

import jax
import jax.numpy as jnp
from jax import lax
from jax.experimental import pallas as pl
from jax.experimental.pallas import tpu as pltpu
from jax.experimental.pallas import tpu_sc as plsc

_D = 64
_L = 200
_B = 4096
_NW = 32
_ROWS = _B * _L
_RPW = _ROWS // _NW
_S = 2
_CH = _S * _L
_NCH = _RPW // _CH
_NV = _D // 16


def _add_pos(rows_v, pos_v):
    def lbody(l, carry):
        pvs = [pos_v[l, pl.ds(c * 16, 16)] for c in range(_NV)]
        for s in range(_S):
            r = s * _L + l
            for c in range(_NV):
                rows_v[r, pl.ds(c * 16, 16)] = (
                    rows_v[r, pl.ds(c * 16, 16)] + pvs[c]
                )
        return carry
    lax.fori_loop(0, _L, lbody, 0, unroll=4)


def _sc_body(x_hbm, tok_hbm, pos_hbm, out_hbm,
             pos_v, idx0, idx1, rows0, rows1,
             isem0, isem1, gsem0, gsem1, wsem0, wsem1):
    wid = lax.axis_index("s") * 2 + lax.axis_index("c")
    wbase = wid * _RPW

    bufs = ((idx0, rows0, isem0, gsem0, wsem0),
            (idx1, rows1, isem1, gsem1, wsem1))

    pltpu.sync_copy(pos_hbm, pos_v)

    def idx_start(g, idx_b, isem_b):
        pltpu.async_copy(x_hbm.at[pl.ds(wbase + g * _CH, _CH)], idx_b, isem_b)

    def idx_wait(g, idx_b, isem_b):
        pltpu.make_async_copy(
            x_hbm.at[pl.ds(wbase + g * _CH, _CH)], idx_b, isem_b).wait()

    def gather_start(idx_b, rows_b, gsem_b):
        pltpu.async_copy(tok_hbm.at[idx_b], rows_b, gsem_b)

    def gather_wait(idx_b, rows_b, gsem_b):
        pltpu.make_async_copy(tok_hbm.at[idx_b], rows_b, gsem_b).wait()

    def write_start(g, rows_b, wsem_b):
        pltpu.async_copy(
            rows_b, out_hbm.at[pl.ds(wbase + g * _CH, _CH), pl.ds(0, _D)],
            wsem_b)

    def write_wait(g, rows_b, wsem_b):
        pltpu.make_async_copy(
            rows_b, out_hbm.at[pl.ds(wbase + g * _CH, _CH), pl.ds(0, _D)],
            wsem_b).wait()

    idx_start(0, idx0, isem0)
    idx_wait(0, idx0, isem0)
    gather_start(idx0, rows0, gsem0)
    idx_start(1, idx1, isem1)

    def outer(i, carry):
        for b in range(2):
            g = i * 2 + b
            idx_b, rows_b, isem_b, gsem_b, wsem_b = bufs[b]
            idx_n, rows_n, isem_n, gsem_n, wsem_n = bufs[1 - b]

            @pl.when(g + 1 < _NCH)
            def _start_next_gather():
                @pl.when(g >= 1)
                def _drain():
                    write_wait(g - 1, rows_n, wsem_n)
                idx_wait(g + 1, idx_n, isem_n)
                gather_start(idx_n, rows_n, gsem_n)

            gather_wait(idx_b, rows_b, gsem_b)

            @pl.when(g + 2 < _NCH)
            def _start_next_idx():
                idx_start(g + 2, idx_b, isem_b)

            _add_pos(rows_b, pos_v)
            write_start(g, rows_b, wsem_b)
        return carry

    lax.fori_loop(0, _NCH // 2, outer, 0)

    write_wait(_NCH - 2, rows0, wsem0)
    write_wait(_NCH - 1, rows1, wsem1)


@jax.jit
def _sc_embed(x, token_table, pos_table):
    mesh = plsc.VectorSubcoreMesh(core_axis_name="c", subcore_axis_name="s")
    out = pl.kernel(
        _sc_body,
        mesh=mesh,
        out_type=jax.ShapeDtypeStruct((_ROWS, 128), jnp.float32),
        compiler_params=pltpu.CompilerParams(use_tc_tiling_on_sc=False),
        scratch_types=[
            pltpu.VMEM((_L, _D), jnp.float32),
            pltpu.VMEM((_CH,), jnp.int32),
            pltpu.VMEM((_CH,), jnp.int32),
            pltpu.VMEM((_CH, _D), jnp.float32),
            pltpu.VMEM((_CH, _D), jnp.float32),
            pltpu.SemaphoreType.DMA,
            pltpu.SemaphoreType.DMA,
            pltpu.SemaphoreType.DMA,
            pltpu.SemaphoreType.DMA,
            pltpu.SemaphoreType.DMA,
            pltpu.SemaphoreType.DMA,
        ],
    )(x.reshape(_ROWS), token_table, pos_table)
    return out[:, :_D].reshape(_B, _L, _D)


def kernel(x, token_table, pos_table):
    return _sc_embed(x, token_table, pos_table)


# speedup vs baseline: 2.1058x; 1.0292x over previous
import functools

import jax
import jax.numpy as jnp
from jax import lax
from jax.experimental import pallas as pl
from jax.experimental.pallas import tpu as pltpu
from jax.experimental.pallas import tpu_sc as plsc

_D = 64
_L = 200
_B = 4096
_NW = 32
_ROWS = _B * _L
_RPW = _ROWS // _NW
_S = 4
_CH = _S * _L
_NCH = _RPW // _CH
_NV = _D // 16


def _add_pos(rows_v, pos_v):
    def lbody(l, carry):
        pvs = [pos_v[l, pl.ds(c * 16, 16)] for c in range(_NV)]
        for s in range(_S):
            r = s * _L + l
            for c in range(_NV):
                rows_v[r, pl.ds(c * 16, 16)] = (
                    rows_v[r, pl.ds(c * 16, 16)] + pvs[c]
                )
        return carry
    lax.fori_loop(0, _L, lbody, 0, unroll=2)


def _sc_body(x_hbm, tok_hbm, pos_hbm, out_hbm,
             pos_v, idx0, idx1, rows0, rows1,
             isem0, isem1, gsem0, gsem1, wsem0, wsem1):
    wid = lax.axis_index("s") * 2 + lax.axis_index("c")
    wbase = wid * _RPW

    bufs = ((idx0, rows0, isem0, gsem0, wsem0),
            (idx1, rows1, isem1, gsem1, wsem1))

    pltpu.sync_copy(pos_hbm, pos_v)

    def idx_start(g, idx_b, isem_b):
        pltpu.async_copy(x_hbm.at[pl.ds(wbase + g * _CH, _CH)], idx_b, isem_b)

    def idx_wait(g, idx_b, isem_b):
        pltpu.make_async_copy(
            x_hbm.at[pl.ds(wbase + g * _CH, _CH)], idx_b, isem_b).wait()

    def gather_start(idx_b, rows_b, gsem_b):
        pltpu.async_copy(tok_hbm.at[idx_b], rows_b, gsem_b)

    def gather_wait(idx_b, rows_b, gsem_b):
        pltpu.make_async_copy(tok_hbm.at[idx_b], rows_b, gsem_b).wait()

    def write_start(g, rows_b, wsem_b):
        pltpu.async_copy(
            rows_b, out_hbm.at[pl.ds(wbase + g * _CH, _CH), pl.ds(0, _D)],
            wsem_b)

    def write_wait(g, rows_b, wsem_b):
        pltpu.make_async_copy(
            rows_b, out_hbm.at[pl.ds(wbase + g * _CH, _CH), pl.ds(0, _D)],
            wsem_b).wait()

    idx_start(0, idx0, isem0)
    idx_wait(0, idx0, isem0)
    gather_start(idx0, rows0, gsem0)
    idx_start(1, idx1, isem1)

    def outer(i, carry):
        for b in range(2):
            g = i * 2 + b
            idx_b, rows_b, isem_b, gsem_b, wsem_b = bufs[b]
            idx_n, rows_n, isem_n, gsem_n, wsem_n = bufs[1 - b]

            @pl.when(g + 1 < _NCH)
            def _start_next_gather():
                @pl.when(g >= 1)
                def _drain():
                    write_wait(g - 1, rows_n, wsem_n)
                idx_wait(g + 1, idx_n, isem_n)
                gather_start(idx_n, rows_n, gsem_n)

            gather_wait(idx_b, rows_b, gsem_b)

            @pl.when(g + 2 < _NCH)
            def _start_next_idx():
                idx_start(g + 2, idx_b, isem_b)

            _add_pos(rows_b, pos_v)
            write_start(g, rows_b, wsem_b)
        return carry

    lax.fori_loop(0, _NCH // 2, outer, 0)

    write_wait(_NCH - 2, rows0, wsem0)
    write_wait(_NCH - 1, rows1, wsem1)


@jax.jit
def _sc_embed(x, token_table, pos_table):
    mesh = plsc.VectorSubcoreMesh(core_axis_name="c", subcore_axis_name="s")
    out = pl.kernel(
        _sc_body,
        mesh=mesh,
        out_type=jax.ShapeDtypeStruct((_ROWS, 128), jnp.float32),
        compiler_params=pltpu.CompilerParams(use_tc_tiling_on_sc=False),
        scratch_types=[
            pltpu.VMEM((_L, _D), jnp.float32),
            pltpu.VMEM((_CH,), jnp.int32),
            pltpu.VMEM((_CH,), jnp.int32),
            pltpu.VMEM((_CH, _D), jnp.float32),
            pltpu.VMEM((_CH, _D), jnp.float32),
            pltpu.SemaphoreType.DMA,
            pltpu.SemaphoreType.DMA,
            pltpu.SemaphoreType.DMA,
            pltpu.SemaphoreType.DMA,
            pltpu.SemaphoreType.DMA,
            pltpu.SemaphoreType.DMA,
        ],
    )(x.reshape(_ROWS), token_table, pos_table)
    return out[:, :_D].reshape(_B, _L, _D)


def kernel(x, token_table, pos_table):
    return _sc_embed(x, token_table, pos_table)


# speedup vs baseline: 2.1138x; 1.0038x over previous
import functools

import jax
import jax.numpy as jnp
from jax import lax
from jax.experimental import pallas as pl
from jax.experimental.pallas import tpu as pltpu
from jax.experimental.pallas import tpu_sc as plsc

_D = 64
_L = 200
_B = 4096
_NW = 32
_ROWS = _B * _L
_RPW = _ROWS // _NW
_S = 4
_CH = _S * _L
_NCH = _RPW // _CH
_NV = _D // 16


def _add_pos(rows_v, pos_v):
    @functools.partial(plsc.parallel_loop, 0, _L, unroll=4)
    def lbody(l):
        pvs = [pos_v[l, pl.ds(c * 16, 16)] for c in range(_NV)]
        for s in range(_S):
            r = s * _L + l
            for c in range(_NV):
                rows_v[r, pl.ds(c * 16, 16)] = (
                    rows_v[r, pl.ds(c * 16, 16)] + pvs[c]
                )


def _sc_body(x_hbm, tok_hbm, pos_hbm, out_hbm,
             pos_v, idx0, idx1, rows0, rows1,
             isem0, isem1, gsem0, gsem1, wsem0, wsem1):
    wid = lax.axis_index("s") * 2 + lax.axis_index("c")
    wbase = wid * _RPW

    bufs = ((idx0, rows0, isem0, gsem0, wsem0),
            (idx1, rows1, isem1, gsem1, wsem1))

    pltpu.sync_copy(pos_hbm, pos_v)

    def idx_start(g, idx_b, isem_b):
        pltpu.async_copy(x_hbm.at[pl.ds(wbase + g * _CH, _CH)], idx_b, isem_b)

    def idx_wait(g, idx_b, isem_b):
        pltpu.make_async_copy(
            x_hbm.at[pl.ds(wbase + g * _CH, _CH)], idx_b, isem_b).wait()

    def gather_start(idx_b, rows_b, gsem_b):
        pltpu.async_copy(tok_hbm.at[idx_b], rows_b, gsem_b)

    def gather_wait(idx_b, rows_b, gsem_b):
        pltpu.make_async_copy(tok_hbm.at[idx_b], rows_b, gsem_b).wait()

    def write_start(g, rows_b, wsem_b):
        pltpu.async_copy(
            rows_b, out_hbm.at[pl.ds(wbase + g * _CH, _CH), pl.ds(0, _D)],
            wsem_b)

    def write_wait(g, rows_b, wsem_b):
        pltpu.make_async_copy(
            rows_b, out_hbm.at[pl.ds(wbase + g * _CH, _CH), pl.ds(0, _D)],
            wsem_b).wait()

    idx_start(0, idx0, isem0)
    idx_wait(0, idx0, isem0)
    gather_start(idx0, rows0, gsem0)
    idx_start(1, idx1, isem1)

    def outer(i, carry):
        for b in range(2):
            g = i * 2 + b
            idx_b, rows_b, isem_b, gsem_b, wsem_b = bufs[b]
            idx_n, rows_n, isem_n, gsem_n, wsem_n = bufs[1 - b]

            @pl.when(g + 1 < _NCH)
            def _start_next_gather():
                @pl.when(g >= 1)
                def _drain():
                    write_wait(g - 1, rows_n, wsem_n)
                idx_wait(g + 1, idx_n, isem_n)
                gather_start(idx_n, rows_n, gsem_n)

            gather_wait(idx_b, rows_b, gsem_b)

            @pl.when(g + 2 < _NCH)
            def _start_next_idx():
                idx_start(g + 2, idx_b, isem_b)

            _add_pos(rows_b, pos_v)
            write_start(g, rows_b, wsem_b)
        return carry

    lax.fori_loop(0, _NCH // 2, outer, 0)

    write_wait(_NCH - 2, rows0, wsem0)
    write_wait(_NCH - 1, rows1, wsem1)


@jax.jit
def _sc_embed(x, token_table, pos_table):
    mesh = plsc.VectorSubcoreMesh(core_axis_name="c", subcore_axis_name="s")
    out = pl.kernel(
        _sc_body,
        mesh=mesh,
        out_type=jax.ShapeDtypeStruct((_ROWS, 128), jnp.float32),
        compiler_params=pltpu.CompilerParams(use_tc_tiling_on_sc=False),
        scratch_types=[
            pltpu.VMEM((_L, _D), jnp.float32),
            pltpu.VMEM((_CH,), jnp.int32),
            pltpu.VMEM((_CH,), jnp.int32),
            pltpu.VMEM((_CH, _D), jnp.float32),
            pltpu.VMEM((_CH, _D), jnp.float32),
            pltpu.SemaphoreType.DMA,
            pltpu.SemaphoreType.DMA,
            pltpu.SemaphoreType.DMA,
            pltpu.SemaphoreType.DMA,
            pltpu.SemaphoreType.DMA,
            pltpu.SemaphoreType.DMA,
        ],
    )(x.reshape(_ROWS), token_table, pos_table)
    return out[:, :_D].reshape(_B, _L, _D)


def kernel(x, token_table, pos_table):
    return _sc_embed(x, token_table, pos_table)
